# baseline (speedup 1.0000x reference)
import functools

import jax
import jax.numpy as jnp
from jax import lax
from jax.experimental import pallas as pl
from jax.experimental.pallas import tpu as pltpu
from jax.experimental.pallas import tpu_sc as plsc

NC = 2
NS = 16
NW = NC * NS
LB = 128
BN = 1024


def _ceil_to(a, m):
    return (a + m - 1) // m * m


def _deg_kernel(n_pad, chunks):
    nr = n_pad // NS

    @functools.partial(
        pl.kernel,
        out_type=jax.ShapeDtypeStruct((NC, n_pad), jnp.float32),
        mesh=plsc.VectorSubcoreMesh(core_axis_name="c", subcore_axis_name="s"),
        scratch_types=[
            pltpu.VMEM((chunks, LB), jnp.int32),
            pltpu.VMEM((LB,), jnp.float32),
            pltpu.VMEM((nr,), jnp.float32),
            pltpu.VMEM_SHARED((n_pad,), jnp.float32),
        ],
    )
    def deg_kernel(dst_hbm, degp_hbm, idx_v, ones_v, wb_v, deg_sh):
        c = lax.axis_index("c")
        s = lax.axis_index("s")
        wid = c * NS + s
        ones16 = jnp.ones((16,), jnp.float32)
        zeros16 = jnp.zeros((16,), jnp.float32)
        for j in range(LB // 16):
            ones_v[pl.ds(j * 16, 16)] = ones16

        def zbody(i, carry):
            wb_v[pl.ds(i * 16, 16)] = zeros16
            return carry

        lax.fori_loop(0, nr // 16, zbody, 0)
        pltpu.sync_copy(wb_v, deg_sh.at[pl.ds(s * nr, nr)])
        pltpu.sync_copy(dst_hbm.at[wid], idx_v)
        plsc.subcore_barrier()

        def body(j, carry):
            pltpu.sync_copy(ones_v, deg_sh.at[idx_v.at[j]], add=True)
            return carry

        lax.fori_loop(0, chunks, body, 0)
        plsc.subcore_barrier()
        pltpu.sync_copy(deg_sh.at[pl.ds(s * nr, nr)], wb_v)
        pltpu.sync_copy(wb_v, degp_hbm.at[c, pl.ds(s * nr, nr)])

    return deg_kernel


def _agg_kernel(n_pad, tch, n_slow, slow_core):
    n_half = n_pad // 2
    nr = n_half // NS
    k_buf = 3
    n_fast = tch // NS - n_slow
    idx_rows = max(n_slow, n_fast)

    @functools.partial(
        pl.kernel,
        out_type=jax.ShapeDtypeStruct((NC, n_half, 128), jnp.float32),
        mesh=plsc.VectorSubcoreMesh(core_axis_name="c", subcore_axis_name="s"),
        scratch_types=[
            pltpu.VMEM((idx_rows, LB), jnp.int32),
            pltpu.VMEM((idx_rows, LB), jnp.int32),
            pltpu.VMEM((k_buf, LB, 128), jnp.float32),
            pltpu.VMEM_SHARED((n_half, 128), jnp.float32),
            pltpu.SemaphoreType.DMA((k_buf,)),
        ],
    )
    def agg_kernel(src_hbm, dst_hbm, hs2_hbm, accp_hbm,
                   isrc_v, idst_v, rows_v, acc_sh, sem):
        c = lax.axis_index("c")
        s = lax.axis_index("s")
        zeros16 = jnp.zeros((16,), jnp.float32)

        def zbody(r, carry):
            for k in range(128 // 16):
                rows_v[0, r, pl.ds(k * 16, 16)] = zeros16
            return carry

        lax.fori_loop(0, 64, zbody, 0)
        for k in range(nr // 64):
            pltpu.sync_copy(rows_v.at[0, pl.ds(0, 64)],
                            acc_sh.at[pl.ds(s * nr + k * 64, 64)])

        def pipeline(wid, npc):
            pltpu.sync_copy(src_hbm.at[wid], isrc_v)
            pltpu.sync_copy(dst_hbm.at[wid], idst_v)
            plsc.subcore_barrier()
            for j in range(k_buf):
                pltpu.async_copy(hs2_hbm.at[isrc_v.at[j]], rows_v.at[j],
                                 sem.at[j])

            def body(j, carry):
                buf = lax.rem(j, k_buf)
                pltpu.make_async_copy(hs2_hbm.at[isrc_v.at[j]], rows_v.at[buf],
                                      sem.at[buf]).wait()
                pltpu.sync_copy(rows_v.at[buf], acc_sh.at[idst_v.at[j]],
                                add=True)

                @pl.when(j + k_buf < npc)
                def _():
                    pltpu.async_copy(hs2_hbm.at[isrc_v.at[j + k_buf]],
                                     rows_v.at[buf], sem.at[buf])

                return carry

            lax.fori_loop(0, npc, body, 0)

        pipeline(c * NS + s, n_slow)

        plsc.subcore_barrier()
        for k in range(nr // 64):
            pltpu.sync_copy(acc_sh.at[pl.ds(s * nr + k * 64, 64)],
                            rows_v.at[0, pl.ds(0, 64)])
            pltpu.sync_copy(rows_v.at[0, pl.ds(0, 64)],
                            accp_hbm.at[c, pl.ds(s * nr + k * 64, 64)])

    return agg_kernel


def _matmul_scale(x_pad, W_pad, degp_t, n_pad):
    nfeat = x_pad.shape[1]

    def body(x_ref, w_ref, degp_ref, hs2_ref):
        degs = degp_ref[...]
        deg = degs[:, 0:1] + degs[:, 1:2] + 1.0
        dis = lax.rsqrt(deg)
        h = jnp.dot(x_ref[...], w_ref[...],
                    preferred_element_type=jnp.float32) * dis
        hs2_ref[:, 0, :] = h
        hs2_ref[:, 1, :] = jnp.concatenate([h[:, 64:], h[:, :64]], axis=1)

    return pl.pallas_call(
        body,
        grid=(n_pad // BN,),
        in_specs=[
            pl.BlockSpec((BN, nfeat), lambda i: (i, 0)),
            pl.BlockSpec((nfeat, 128), lambda i: (0, 0)),
            pl.BlockSpec((BN, NC), lambda i: (i, 0)),
        ],
        out_specs=pl.BlockSpec((BN, 2, 128), lambda i: (i, 0, 0)),
        out_shape=jax.ShapeDtypeStruct((n_pad, 2, 128), jnp.float32),
    )(x_pad, W_pad, degp_t)


def _combine(degp_t, accp64, hs2, n_pad, ncol):
    def body(degp_ref, accp_ref, hs2_ref, out_ref):
        degs = degp_ref[...]
        deg = degs[:, 0:1] + degs[:, 1:2] + 1.0
        dis = lax.rsqrt(deg)
        tot = accp_ref[0] + accp_ref[1] + hs2_ref[:, 0, :ncol]
        out_ref[...] = tot * dis

    return pl.pallas_call(
        body,
        grid=(n_pad // BN,),
        in_specs=[
            pl.BlockSpec((BN, NC), lambda i: (i, 0)),
            pl.BlockSpec((NC, BN, ncol), lambda i: (0, i, 0)),
            pl.BlockSpec((BN, 2, 128), lambda i: (i, 0, 0)),
        ],
        out_specs=pl.BlockSpec((BN, ncol), lambda i: (i, 0)),
        out_shape=jax.ShapeDtypeStruct((n_pad, ncol), jnp.float32),
    )(degp_t, accp64, hs2)


def kernel(x, edge_index, W):
    n = x.shape[0]
    e = edge_index.shape[1]
    n_pad = _ceil_to(n, BN * 2)
    e_pad = _ceil_to(e, NS * LB * 8)
    chunks = e_pad // (NW * LB)

    ei = edge_index.astype(jnp.int32)
    pad_e = e_pad - e
    src = jnp.concatenate([ei[0], jnp.full((pad_e,), n, dtype=jnp.int32)])
    dst = jnp.concatenate([ei[1], jnp.full((pad_e,), n, dtype=jnp.int32)])
    tch = e_pad // LB
    gidx = (src * 2 + (dst & 1)).reshape(NW, chunks, LB)
    sidx = (dst >> 1).reshape(NW, chunks, LB)
    dst3 = dst.reshape(NW, chunks, LB)
    x_pad = jnp.pad(x, ((0, n_pad - n), (0, 0)))

    ncol = W.shape[1]
    W_pad = jnp.pad(W, ((0, 0), (0, 128 - ncol)))

    degp = _deg_kernel(n_pad, chunks)(dst3)
    degp_t = degp.T
    hs2 = _matmul_scale(x_pad, W_pad, degp_t, n_pad)
    hs2_flat = hs2.reshape(2 * n_pad, 128)
    n_slow = tch // NS // 2
    accp = _agg_kernel(n_pad, tch, n_slow, 1)(gidx, sidx, hs2_flat)
    accp64 = accp.reshape(NC, n_pad, 64)
    out = _combine(degp_t, accp64, hs2, n_pad, ncol)
    return out[:n]


# speedup vs baseline: 3.0391x; 3.0391x over previous
import functools

import jax
import jax.numpy as jnp
from jax import lax
from jax.experimental import pallas as pl
from jax.experimental.pallas import tpu as pltpu
from jax.experimental.pallas import tpu_sc as plsc

NC = 2
NS = 16
NW = NC * NS
LB = 128
BN = 1024


def _ceil_to(a, m):
    return (a + m - 1) // m * m


def _deg_kernel(n_pad, chunks):
    nr = n_pad // NS

    @functools.partial(
        pl.kernel,
        out_type=jax.ShapeDtypeStruct((NC, n_pad), jnp.float32),
        mesh=plsc.VectorSubcoreMesh(core_axis_name="c", subcore_axis_name="s"),
        scratch_types=[
            pltpu.VMEM((chunks, LB), jnp.int32),
            pltpu.VMEM((LB,), jnp.float32),
            pltpu.VMEM((nr,), jnp.float32),
            pltpu.VMEM_SHARED((n_pad,), jnp.float32),
        ],
    )
    def deg_kernel(dst_hbm, degp_hbm, idx_v, ones_v, wb_v, deg_sh):
        c = lax.axis_index("c")
        s = lax.axis_index("s")
        wid = c * NS + s
        ones16 = jnp.ones((16,), jnp.float32)
        zeros16 = jnp.zeros((16,), jnp.float32)
        for j in range(LB // 16):
            ones_v[pl.ds(j * 16, 16)] = ones16

        def zbody(i, carry):
            wb_v[pl.ds(i * 16, 16)] = zeros16
            return carry

        lax.fori_loop(0, nr // 16, zbody, 0)
        pltpu.sync_copy(wb_v, deg_sh.at[pl.ds(s * nr, nr)])
        pltpu.sync_copy(dst_hbm.at[wid], idx_v)
        plsc.subcore_barrier()

        def body(j, carry):
            pltpu.sync_copy(ones_v, deg_sh.at[idx_v.at[j]], add=True)
            return carry

        lax.fori_loop(0, chunks, body, 0)
        plsc.subcore_barrier()
        pltpu.sync_copy(deg_sh.at[pl.ds(s * nr, nr)], wb_v)
        pltpu.sync_copy(wb_v, degp_hbm.at[c, pl.ds(s * nr, nr)])

    return deg_kernel


def _agg_kernel(n_pad, tch, n_slow, slow_core):
    n_half = n_pad // 2
    nr = n_half // NS
    k_buf = 3
    n_fast = tch // NS - n_slow
    idx_rows = max(n_slow, n_fast)

    @functools.partial(
        pl.kernel,
        out_type=jax.ShapeDtypeStruct((NC, n_half, 128), jnp.float32),
        mesh=plsc.VectorSubcoreMesh(core_axis_name="c", subcore_axis_name="s"),
        scratch_types=[
            pltpu.VMEM((idx_rows, LB), jnp.int32),
            pltpu.VMEM((idx_rows, LB), jnp.int32),
            pltpu.VMEM((k_buf, LB, 128), jnp.float32),
            pltpu.VMEM_SHARED((n_half, 128), jnp.float32),
            pltpu.SemaphoreType.DMA((k_buf,)),
        ],
    )
    def agg_kernel(src_hbm, dst_hbm, hs2_hbm, accp_hbm,
                   isrc_v, idst_v, rows_v, acc_sh, sem):
        c = lax.axis_index("c")
        s = lax.axis_index("s")
        zeros16 = jnp.zeros((16,), jnp.float32)

        def zbody(r, carry):
            for k in range(128 // 16):
                rows_v[0, r, pl.ds(k * 16, 16)] = zeros16
            return carry

        lax.fori_loop(0, 64, zbody, 0)
        for k in range(nr // 64):
            pltpu.sync_copy(rows_v.at[0, pl.ds(0, 64)],
                            acc_sh.at[pl.ds(s * nr + k * 64, 64)])

        def pipeline(wid, npc):
            pltpu.sync_copy(src_hbm.at[wid], isrc_v)
            pltpu.sync_copy(dst_hbm.at[wid], idst_v)
            plsc.subcore_barrier()
            for j in range(k_buf):
                pltpu.async_copy(hs2_hbm.at[isrc_v.at[j]], rows_v.at[j],
                                 sem.at[j])

            def body(j, carry):
                buf = lax.rem(j, k_buf)
                pltpu.make_async_copy(hs2_hbm.at[isrc_v.at[j]], rows_v.at[buf],
                                      sem.at[buf]).wait()
                pltpu.sync_copy(rows_v.at[buf], acc_sh.at[idst_v.at[j]],
                                add=True)

                @pl.when(j + k_buf < npc)
                def _():
                    pltpu.async_copy(hs2_hbm.at[isrc_v.at[j + k_buf]],
                                     rows_v.at[buf], sem.at[buf])

                return carry

            lax.fori_loop(0, npc, body, 0)

        pipeline(c * NS + s, n_slow)

        plsc.subcore_barrier()
        for k in range(nr // 64):
            pltpu.sync_copy(acc_sh.at[pl.ds(s * nr + k * 64, 64)],
                            rows_v.at[0, pl.ds(0, 64)])
            pltpu.sync_copy(rows_v.at[0, pl.ds(0, 64)],
                            accp_hbm.at[c, pl.ds(s * nr + k * 64, 64)])

    return agg_kernel


def _matmul_scale(x_pad, W_pad, degp_t, n_pad):
    nfeat = x_pad.shape[1]

    def body(x_ref, w_ref, degp_ref, hs2_ref):
        degs = degp_ref[...]
        deg = degs[:, 0:1] + degs[:, 1:2] + 1.0
        dis = lax.rsqrt(deg)
        h = jnp.dot(x_ref[...], w_ref[...],
                    preferred_element_type=jnp.float32) * dis
        hs2_ref[:, 0, :] = h
        hs2_ref[:, 1, :] = jnp.concatenate([h[:, 64:], h[:, :64]], axis=1)

    return pl.pallas_call(
        body,
        grid=(n_pad // BN,),
        in_specs=[
            pl.BlockSpec((BN, nfeat), lambda i: (i, 0)),
            pl.BlockSpec((nfeat, 128), lambda i: (0, 0)),
            pl.BlockSpec((BN, NC), lambda i: (i, 0)),
        ],
        out_specs=pl.BlockSpec((BN, 2, 128), lambda i: (i, 0, 0)),
        out_shape=jax.ShapeDtypeStruct((n_pad, 2, 128), jnp.float32),
    )(x_pad, W_pad, degp_t)


def _combine(degp_t, accp64, hs2, n_pad, ncol):
    def body(degp_ref, accp_ref, hs2_ref, out_ref):
        degs = degp_ref[...]
        deg = degs[:, 0:1] + degs[:, 1:2] + 1.0
        dis = lax.rsqrt(deg)
        tot = accp_ref[0] + accp_ref[1] + hs2_ref[:, 0, :ncol]
        out_ref[...] = tot * dis

    return pl.pallas_call(
        body,
        grid=(n_pad // BN,),
        in_specs=[
            pl.BlockSpec((BN, NC), lambda i: (i, 0)),
            pl.BlockSpec((NC, BN, ncol), lambda i: (0, i, 0)),
            pl.BlockSpec((BN, 2, 128), lambda i: (i, 0, 0)),
        ],
        out_specs=pl.BlockSpec((BN, ncol), lambda i: (i, 0)),
        out_shape=jax.ShapeDtypeStruct((n_pad, ncol), jnp.float32),
    )(degp_t, accp64, hs2)


def kernel(x, edge_index, W):
    n = x.shape[0]
    e = edge_index.shape[1]
    n_pad = _ceil_to(n, BN * 2)
    e_pad = _ceil_to(e, NS * LB * 8)
    chunks = e_pad // (NW * LB)

    ei = edge_index.astype(jnp.int32)
    pad_e = e_pad - e
    trash = n + jnp.arange(pad_e, dtype=jnp.int32) % (n_pad - n)
    src = jnp.concatenate([ei[0], trash])
    dst = jnp.concatenate([ei[1], trash])
    tch = e_pad // LB
    gidx = (src * 2 + (dst & 1)).reshape(NW, chunks, LB)
    sidx = (dst >> 1).reshape(NW, chunks, LB)
    dst3 = dst.reshape(NW, chunks, LB)
    x_pad = jnp.pad(x, ((0, n_pad - n), (0, 0)))

    ncol = W.shape[1]
    W_pad = jnp.pad(W, ((0, 0), (0, 128 - ncol)))

    degp = _deg_kernel(n_pad, chunks)(dst3)
    degp_t = degp.T
    hs2 = _matmul_scale(x_pad, W_pad, degp_t, n_pad)
    hs2_flat = hs2.reshape(2 * n_pad, 128)
    n_slow = tch // NS // 2
    accp = _agg_kernel(n_pad, tch, n_slow, 1)(gidx, sidx, hs2_flat)
    accp64 = accp.reshape(NC, n_pad, 64)
    out = _combine(degp_t, accp64, hs2, n_pad, ncol)
    return out[:n]
